# baseline (speedup 1.0000x reference)
import jax
import jax.numpy as jnp
from jax import lax
from jax.experimental import pallas as pl
from jax.experimental.pallas import tpu as pltpu
from jax.experimental.pallas import tpu_sc as plsc

N = 10000
E = 320000
D = 128
NC, NS = 2, 16
NW = NC * NS
CH = 128
CPW = 80
NCHUNK = CPW * NW
EPAD = NCHUNK * CH
NPAD = 10240
ZROWS = NPAD // NS
NAGG = 7168
DUMP = 5000
ZAGG = NAGG // NS
OROWS = 320
OPAD = OROWS * NS

_mesh = plsc.VectorSubcoreMesh(core_axis_name="c", subcore_axis_name="s")


def _worker_id():
    return lax.axis_index("c") * NS + lax.axis_index("s")


def _hist_body(rows_hbm, deg_out, deg_sh, idx_v, ones_v, zero_v):
    c = lax.axis_index("c")
    s = lax.axis_index("s")
    wid = c * NS + s

    lanes = lax.broadcasted_iota(jnp.int32, (16,), 0)
    one_row = jnp.where(lanes == 0, 1.0, 0.0).astype(jnp.float32)
    zrow = jnp.zeros((16,), jnp.float32)

    def fill(r, _):
        ones_v[r, pl.ds(0, 16)] = one_row
        for q in range(1, D // 16):
            ones_v[r, pl.ds(q * 16, 16)] = zrow
        for q in range(D // 16):
            zero_v[r, pl.ds(q * 16, 16)] = zrow
        return 0

    lax.fori_loop(0, CH, fill, 0)

    pltpu.sync_copy(rows_hbm.at[pl.ds(wid * CPW, CPW)], idx_v)

    base = s * ZROWS
    for k in range(ZROWS // CH):
        pltpu.sync_copy(zero_v, deg_sh.at[pl.ds(base + k * CH, CH)])
    plsc.subcore_barrier()

    def scat(j, _):
        pltpu.sync_copy(ones_v, deg_sh.at[idx_v.at[j]], add=True)
        return 0

    lax.fori_loop(0, CPW, scat, 0)
    plsc.subcore_barrier()
    pltpu.sync_copy(deg_sh.at[pl.ds(base, ZROWS)],
                    deg_out.at[c, pl.ds(base, ZROWS)])


_hist = pl.kernel(
    _hist_body,
    out_type=jax.ShapeDtypeStruct((NC, NPAD, D), jnp.float32),
    mesh=_mesh,
    scratch_types=[
        pltpu.VMEM_SHARED((NPAD, D), jnp.float32),
        pltpu.VMEM((CPW, CH), jnp.int32),
        pltpu.VMEM((CH, D), jnp.float32),
        pltpu.VMEM((CH, D), jnp.float32),
    ],
)


def _agg_body(xs_hbm, rows_hbm, cols_hbm, part_out,
              agg_sh, rows_v, cols_v, gbuf, zbuf, sem_a, sem_b):
    c = lax.axis_index("c")
    s = lax.axis_index("s")
    wid = c * NS + s
    cb = wid * CPW

    zrow = jnp.zeros((16,), jnp.float32)

    def fill(r, _):
        for q in range(D // 16):
            zbuf[r, pl.ds(q * 16, 16)] = zrow
        return 0

    lax.fori_loop(0, CH, fill, 0)

    pltpu.sync_copy(rows_hbm.at[pl.ds(cb, CPW)], rows_v)
    pltpu.sync_copy(cols_hbm.at[pl.ds(cb, CPW)], cols_v)

    def clamp(j, _):
        for q in range(CH // 16):
            v = cols_v[j, pl.ds(q * 16, 16)]
            dumped = DUMP + ((v - DUMP) & 2047)
            cols_v[j, pl.ds(q * 16, 16)] = jnp.where(v < DUMP, v, dumped)
        return 0

    lax.fori_loop(0, CPW, clamp, 0)

    base = s * ZAGG
    for k in range(ZAGG // CH):
        pltpu.sync_copy(zbuf, agg_sh.at[pl.ds(base + k * CH, CH)])
    rem = ZAGG % CH
    if rem:
        pltpu.sync_copy(zbuf.at[pl.ds(0, rem)],
                        agg_sh.at[pl.ds(base + (ZAGG // CH) * CH, rem)])
    plsc.subcore_barrier()

    def start(j, b, sem):
        pltpu.async_copy(xs_hbm.at[rows_v.at[j]], gbuf.at[b], sem)

    def wait(b, sem):
        pltpu.make_async_copy(xs_hbm.at[rows_v.at[0]], gbuf.at[b], sem).wait()

    def scat(j, b):
        pltpu.sync_copy(gbuf.at[b], agg_sh.at[cols_v.at[j]], add=True)

    start(0, 0, sem_a)

    def body(g, _):
        j0 = 2 * g
        start(j0 + 1, 1, sem_b)
        wait(0, sem_a)
        scat(j0, 0)
        start(j0 + 2, 0, sem_a)
        wait(1, sem_b)
        scat(j0 + 1, 1)
        return 0

    lax.fori_loop(0, CPW // 2 - 1, body, 0)
    start(CPW - 1, 1, sem_b)
    wait(0, sem_a)
    scat(CPW - 2, 0)
    wait(1, sem_b)
    scat(CPW - 1, 1)

    plsc.subcore_barrier()
    pltpu.sync_copy(agg_sh.at[pl.ds(s * OROWS, OROWS)],
                    part_out.at[c, pl.ds(s * OROWS, OROWS)])


_agg = pl.kernel(
    _agg_body,
    out_type=jax.ShapeDtypeStruct((NC, OPAD, D), jnp.float32),
    mesh=_mesh,
    scratch_types=[
        pltpu.VMEM_SHARED((NAGG, D), jnp.float32),
        pltpu.VMEM((CPW, CH), jnp.int32),
        pltpu.VMEM((CPW, CH), jnp.int32),
        pltpu.VMEM((2, CH, D), jnp.float32),
        pltpu.VMEM((CH, D), jnp.float32),
        pltpu.SemaphoreType.DMA,
        pltpu.SemaphoreType.DMA,
    ],
)


def _prescale_body(deg_ref, x_ref, xs_ref):
    deg = deg_ref[0, :, 0:1] + deg_ref[1, :, 0:1]
    dinv = jnp.where(deg > 0, lax.rsqrt(deg), 0.0)
    xs_ref[...] = x_ref[...] * dinv


def _finalize_body(x_ref, deg_ref, part_ref, out_ref):
    deg = deg_ref[0, 0:5000, 0:1] + deg_ref[1, 0:5000, 0:1]
    dinv = jnp.where(deg > 0, lax.rsqrt(deg), 0.0)
    agg = part_ref[0, 0:5000, :] + part_ref[1, 0:5000, :]
    out_ref[:, 0:D] = x_ref[0:5000, :]
    out_ref[:, D:] = agg * dinv


def kernel(x, edge_index, batch_size):
    del batch_size
    arange_pad = jnp.arange(EPAD - E, dtype=jnp.int32)
    pad = jnp.stack([N + arange_pad % (NPAD - N), N + arange_pad % 2048])
    ei = jnp.concatenate([edge_index, pad], axis=1).reshape(2, NCHUNK, CH)
    rows2, cols2 = ei[0], ei[1]
    x_pad = jnp.pad(x, ((0, NPAD - N), (0, 0)))

    deg2 = _hist(rows2)
    xs = pl.pallas_call(
        _prescale_body,
        out_shape=jax.ShapeDtypeStruct((NPAD, D), jnp.float32),
    )(deg2, x_pad)
    part = _agg(xs, rows2, cols2)
    out = pl.pallas_call(
        _finalize_body,
        out_shape=jax.ShapeDtypeStruct((5000, 2 * D), jnp.float32),
    )(x, deg2, part)
    return out


# speedup vs baseline: 1.0053x; 1.0053x over previous
import jax
import jax.numpy as jnp
from jax import lax
from jax.experimental import pallas as pl
from jax.experimental.pallas import tpu as pltpu
from jax.experimental.pallas import tpu_sc as plsc

N = 10000
E = 320000
D = 128
NC, NS = 2, 16
NW = NC * NS
CH = 128
CPW = 80
NCHUNK = CPW * NW
EPAD = NCHUNK * CH
NPAD = 10240
ZROWS = NPAD // NS
NAGG = 7168
DUMP = 5000
ZAGG = NAGG // NS
OROWS = 320
OPAD = OROWS * NS

_mesh = plsc.VectorSubcoreMesh(core_axis_name="c", subcore_axis_name="s")


def _worker_id():
    return lax.axis_index("c") * NS + lax.axis_index("s")


def _hist_body(ei_hbm, deg_out, deg_sh, idx_v, ones_v, zero_v):
    c = lax.axis_index("c")
    s = lax.axis_index("s")
    wid = c * NS + s

    lanes = lax.broadcasted_iota(jnp.int32, (16,), 0)
    one_row = jnp.where(lanes == 0, 1.0, 0.0).astype(jnp.float32)
    zrow = jnp.zeros((16,), jnp.float32)

    def fill(r, _):
        ones_v[r, pl.ds(0, 16)] = one_row
        for q in range(1, D // 16):
            ones_v[r, pl.ds(q * 16, 16)] = zrow
        for q in range(D // 16):
            zero_v[r, pl.ds(q * 16, 16)] = zrow
        return 0

    lax.fori_loop(0, CH, fill, 0)

    pltpu.sync_copy(ei_hbm.at[0, pl.ds(wid * CPW, CPW)], idx_v)

    base = s * ZROWS
    for k in range(ZROWS // CH):
        pltpu.sync_copy(zero_v, deg_sh.at[pl.ds(base + k * CH, CH)])
    plsc.subcore_barrier()

    def scat(j, _):
        pltpu.sync_copy(ones_v, deg_sh.at[idx_v.at[j]], add=True)
        return 0

    lax.fori_loop(0, CPW, scat, 0)
    plsc.subcore_barrier()
    pltpu.sync_copy(deg_sh.at[pl.ds(base, ZROWS)],
                    deg_out.at[c, pl.ds(base, ZROWS)])


_hist = pl.kernel(
    _hist_body,
    out_type=jax.ShapeDtypeStruct((NC, NPAD, D), jnp.float32),
    mesh=_mesh,
    scratch_types=[
        pltpu.VMEM_SHARED((NPAD, D), jnp.float32),
        pltpu.VMEM((CPW, CH), jnp.int32),
        pltpu.VMEM((CH, D), jnp.float32),
        pltpu.VMEM((CH, D), jnp.float32),
    ],
)


def _agg_body(xs_hbm, ei_hbm, part_out,
              agg_sh, rows_v, cols_v, gbuf, zbuf, sem_a, sem_b):
    c = lax.axis_index("c")
    s = lax.axis_index("s")
    wid = c * NS + s
    cb = wid * CPW

    zrow = jnp.zeros((16,), jnp.float32)

    def fill(r, _):
        for q in range(D // 16):
            zbuf[r, pl.ds(q * 16, 16)] = zrow
        return 0

    lax.fori_loop(0, CH, fill, 0)

    pltpu.sync_copy(ei_hbm.at[0, pl.ds(cb, CPW)], rows_v)
    pltpu.sync_copy(ei_hbm.at[1, pl.ds(cb, CPW)], cols_v)

    def clamp(j, _):
        for q in range(CH // 16):
            v = cols_v[j, pl.ds(q * 16, 16)]
            dumped = DUMP + ((v - DUMP) & 2047)
            cols_v[j, pl.ds(q * 16, 16)] = jnp.where(v < DUMP, v, dumped)
        return 0

    lax.fori_loop(0, CPW, clamp, 0)

    base = s * ZAGG
    for k in range(ZAGG // CH):
        pltpu.sync_copy(zbuf, agg_sh.at[pl.ds(base + k * CH, CH)])
    rem = ZAGG % CH
    if rem:
        pltpu.sync_copy(zbuf.at[pl.ds(0, rem)],
                        agg_sh.at[pl.ds(base + (ZAGG // CH) * CH, rem)])
    plsc.subcore_barrier()

    def start(j, b, sem):
        pltpu.async_copy(xs_hbm.at[rows_v.at[j]], gbuf.at[b], sem)

    def wait(b, sem):
        pltpu.make_async_copy(xs_hbm.at[rows_v.at[0]], gbuf.at[b], sem).wait()

    def scat(j, b):
        pltpu.sync_copy(gbuf.at[b], agg_sh.at[cols_v.at[j]], add=True)

    start(0, 0, sem_a)

    def body(g, _):
        j0 = 2 * g
        start(j0 + 1, 1, sem_b)
        wait(0, sem_a)
        scat(j0, 0)
        start(j0 + 2, 0, sem_a)
        wait(1, sem_b)
        scat(j0 + 1, 1)
        return 0

    lax.fori_loop(0, CPW // 2 - 1, body, 0)
    start(CPW - 1, 1, sem_b)
    wait(0, sem_a)
    scat(CPW - 2, 0)
    wait(1, sem_b)
    scat(CPW - 1, 1)

    plsc.subcore_barrier()
    pltpu.sync_copy(agg_sh.at[pl.ds(s * OROWS, OROWS)],
                    part_out.at[c, pl.ds(s * OROWS, OROWS)])


_agg = pl.kernel(
    _agg_body,
    out_type=jax.ShapeDtypeStruct((NC, OPAD, D), jnp.float32),
    mesh=_mesh,
    scratch_types=[
        pltpu.VMEM_SHARED((NAGG, D), jnp.float32),
        pltpu.VMEM((CPW, CH), jnp.int32),
        pltpu.VMEM((CPW, CH), jnp.int32),
        pltpu.VMEM((2, CH, D), jnp.float32),
        pltpu.VMEM((CH, D), jnp.float32),
        pltpu.SemaphoreType.DMA,
        pltpu.SemaphoreType.DMA,
    ],
)


def _prescale_body(deg_ref, x_ref, xs_ref):
    deg = deg_ref[0, 0:N, 0:1] + deg_ref[1, 0:N, 0:1]
    dinv = jnp.where(deg > 0, lax.rsqrt(deg), 0.0)
    xs_ref[0:N, :] = x_ref[...] * dinv
    xs_ref[N:, :] = jnp.zeros((NPAD - N, D), jnp.float32)


def _finalize_body(x_ref, deg_ref, part_ref, out_ref):
    deg = deg_ref[0, 0:5000, 0:1] + deg_ref[1, 0:5000, 0:1]
    dinv = jnp.where(deg > 0, lax.rsqrt(deg), 0.0)
    agg = part_ref[0, 0:5000, :] + part_ref[1, 0:5000, :]
    out_ref[:, 0:D] = x_ref[0:5000, :]
    out_ref[:, D:] = agg * dinv


def kernel(x, edge_index, batch_size):
    del batch_size
    arange_pad = jnp.arange(EPAD - E, dtype=jnp.int32)
    pad = jnp.stack([N + arange_pad % (NPAD - N), N + arange_pad % 2048])
    ei = jnp.concatenate([edge_index, pad], axis=1).reshape(2, NCHUNK, CH)

    deg2 = _hist(ei)
    xs = pl.pallas_call(
        _prescale_body,
        out_shape=jax.ShapeDtypeStruct((NPAD, D), jnp.float32),
    )(deg2, x)
    part = _agg(xs, ei)
    out = pl.pallas_call(
        _finalize_body,
        out_shape=jax.ShapeDtypeStruct((5000, 2 * D), jnp.float32),
    )(x, deg2, part)
    return out


# speedup vs baseline: 1.0785x; 1.0728x over previous
import jax
import jax.numpy as jnp
from jax import lax
from jax.experimental import pallas as pl
from jax.experimental.pallas import tpu as pltpu
from jax.experimental.pallas import tpu_sc as plsc

N = 10000
E = 320000
D = 128
NC, NS = 2, 16
NW = NC * NS
CH = 128
CPW = 80
NCHUNK = CPW * NW
EPAD = NCHUNK * CH
NPAD = 10240
ZROWS = NPAD // NS
NAGG = 5120
DUMP = 5000
ZAGG = NAGG // NS
OROWS = 320
OPAD = OROWS * NS

_mesh = plsc.VectorSubcoreMesh(core_axis_name="c", subcore_axis_name="s")


def _worker_id():
    return lax.axis_index("c") * NS + lax.axis_index("s")


HW = 128
def _hist_body(ei_hbm, deg_out, deg_sh, idx_v, ones_v, zero_v):
    c = lax.axis_index("c")
    s = lax.axis_index("s")
    wid = c * NS + s

    lanes = lax.broadcasted_iota(jnp.int32, (16,), 0)
    one_row = jnp.where(lanes == 0, 1.0, 0.0).astype(jnp.float32)
    zrow = jnp.zeros((16,), jnp.float32)

    def fill(r, _):
        ones_v[r, pl.ds(0, 16)] = one_row
        for q in range(1, HW // 16):
            ones_v[r, pl.ds(q * 16, 16)] = zrow
        for q in range(HW // 16):
            zero_v[r, pl.ds(q * 16, 16)] = zrow
        return 0

    lax.fori_loop(0, CH, fill, 0)

    pltpu.sync_copy(ei_hbm.at[0, pl.ds(wid * CPW, CPW)], idx_v)

    base = s * ZROWS
    for k in range(ZROWS // CH):
        pltpu.sync_copy(zero_v, deg_sh.at[pl.ds(base + k * CH, CH)])
    plsc.subcore_barrier()

    def scat(j, _):
        pltpu.sync_copy(ones_v, deg_sh.at[idx_v.at[j]], add=True)
        return 0

    lax.fori_loop(0, CPW, scat, 0)
    plsc.subcore_barrier()
    pltpu.sync_copy(deg_sh.at[pl.ds(base, ZROWS)],
                    deg_out.at[c, pl.ds(base, ZROWS)])


_hist = pl.kernel(
    _hist_body,
    out_type=jax.ShapeDtypeStruct((NC, NPAD, HW), jnp.float32),
    mesh=_mesh,
    scratch_types=[
        pltpu.VMEM_SHARED((NPAD, HW), jnp.float32),
        pltpu.VMEM((CPW, CH), jnp.int32),
        pltpu.VMEM((CH, HW), jnp.float32),
        pltpu.VMEM((CH, HW), jnp.float32),
    ],
)


def _agg_body(xs_hbm, ei_hbm, part_out,
              agg_sh, rows_v, cols_v, gbuf, s0, s1, s2, s3):
    c = lax.axis_index("c")
    s = lax.axis_index("s")
    wid = c * NS + s
    cb = wid * CPW

    zrow = jnp.zeros((16,), jnp.float32)

    def fill(r, _):
        for q in range(D // 16):
            gbuf[0, r, pl.ds(q * 16, 16)] = zrow
        return 0

    lax.fori_loop(0, CH, fill, 0)

    pltpu.sync_copy(ei_hbm.at[0, pl.ds(cb, CPW)], rows_v)
    pltpu.sync_copy(ei_hbm.at[1, pl.ds(cb, CPW)], cols_v)

    def clamp(j, _):
        for q in range(CH // 16):
            v = cols_v[j, pl.ds(q * 16, 16)]
            dumped = DUMP + ((v - DUMP) & 63)
            cols_v[j, pl.ds(q * 16, 16)] = jnp.where(v < DUMP, v, dumped)
        return 0

    lax.fori_loop(0, CPW, clamp, 0)

    base = s * ZAGG
    for k in range(ZAGG // CH):
        pltpu.sync_copy(gbuf.at[0], agg_sh.at[pl.ds(base + k * CH, CH)])
    rem = ZAGG % CH
    if rem:
        pltpu.sync_copy(gbuf.at[0, pl.ds(0, rem)],
                        agg_sh.at[pl.ds(base + (ZAGG // CH) * CH, rem)])
    plsc.subcore_barrier()

    def start(j, b, sem):
        pltpu.async_copy(xs_hbm.at[rows_v.at[j]], gbuf.at[b], sem)

    def wait(b, sem):
        pltpu.make_async_copy(xs_hbm.at[rows_v.at[0]], gbuf.at[b], sem).wait()

    def scat(j, b):
        pltpu.sync_copy(gbuf.at[b], agg_sh.at[cols_v.at[j]], add=True)

    sems = (s0, s1, s2, s3)
    for b in range(4):
        start(b, b, sems[b])

    def body(g, _):
        j0 = 4 * g
        for b in range(4):
            wait(b, sems[b])
            scat(j0 + b, b)
            start(j0 + b + 4, b, sems[b])
        return 0

    lax.fori_loop(0, CPW // 4 - 1, body, 0)
    for b in range(4):
        wait(b, sems[b])
        scat(CPW - 4 + b, b)

    plsc.subcore_barrier()
    pltpu.sync_copy(agg_sh.at[pl.ds(s * OROWS, OROWS)],
                    part_out.at[c, pl.ds(s * OROWS, OROWS)])


_agg = pl.kernel(
    _agg_body,
    out_type=jax.ShapeDtypeStruct((NC, OPAD, D), jnp.float32),
    mesh=_mesh,
    scratch_types=[
        pltpu.VMEM_SHARED((NAGG, D), jnp.float32),
        pltpu.VMEM((CPW, CH), jnp.int32),
        pltpu.VMEM((CPW, CH), jnp.int32),
        pltpu.VMEM((4, CH, D), jnp.float32),
        pltpu.SemaphoreType.DMA,
        pltpu.SemaphoreType.DMA,
        pltpu.SemaphoreType.DMA,
        pltpu.SemaphoreType.DMA,
    ],
)


def _prescale_body(deg_ref, x_ref, xs_ref):
    deg = deg_ref[0, 0:N, 0:1] + deg_ref[1, 0:N, 0:1]
    dinv = jnp.where(deg > 0, lax.rsqrt(deg), 0.0)
    xs_ref[0:N, :] = x_ref[...] * dinv
    xs_ref[N:, :] = jnp.zeros((NPAD - N, D), jnp.float32)


def _finalize_body(x_ref, deg_ref, part_ref, out_ref):
    deg = deg_ref[0, 0:5000, 0:1] + deg_ref[1, 0:5000, 0:1]
    dinv = jnp.where(deg > 0, lax.rsqrt(deg), 0.0)
    agg = part_ref[0, 0:5000, :] + part_ref[1, 0:5000, :]
    out_ref[:, 0:D] = x_ref[0:5000, :]
    out_ref[:, D:] = agg * dinv


def kernel(x, edge_index, batch_size):
    del batch_size
    arange_pad = jnp.arange(EPAD - E, dtype=jnp.int32)
    pad = jnp.stack([N + arange_pad % (NPAD - N), N + arange_pad % 64])
    ei = jnp.concatenate([edge_index, pad], axis=1).reshape(2, NCHUNK, CH)

    deg2 = _hist(ei)
    xs = pl.pallas_call(
        _prescale_body,
        out_shape=jax.ShapeDtypeStruct((NPAD, D), jnp.float32),
    )(deg2, x)
    part = _agg(xs, ei)
    out = pl.pallas_call(
        _finalize_body,
        out_shape=jax.ShapeDtypeStruct((5000, 2 * D), jnp.float32),
    )(x, deg2, part)
    return out


# speedup vs baseline: 1.0787x; 1.0001x over previous
import jax
import jax.numpy as jnp
from jax import lax
from jax.experimental import pallas as pl
from jax.experimental.pallas import tpu as pltpu
from jax.experimental.pallas import tpu_sc as plsc

N = 10000
E = 320000
D = 128
NC, NS = 2, 16
NW = NC * NS
CH = 128
CPW = 80
NCHUNK = CPW * NW
EPAD = NCHUNK * CH
NPAD = 10240
ZROWS = NPAD // NS
NAGG = 5120
DUMP = 5000
ZAGG = NAGG // NS
OROWS = 320
OPAD = OROWS * NS

_mesh = plsc.VectorSubcoreMesh(core_axis_name="c", subcore_axis_name="s")


HW = 128
def _hist_body(ei_hbm, deg_out, deg_sh, idx_v, ones_v, zero_v):
    c = lax.axis_index("c")
    s = lax.axis_index("s")
    wid = c * NS + s

    lanes = lax.broadcasted_iota(jnp.int32, (16,), 0)
    one_row = jnp.where(lanes == 0, 1.0, 0.0).astype(jnp.float32)
    zrow = jnp.zeros((16,), jnp.float32)

    def fill(r, _):
        ones_v[r, pl.ds(0, 16)] = one_row
        for q in range(1, HW // 16):
            ones_v[r, pl.ds(q * 16, 16)] = zrow
        for q in range(HW // 16):
            zero_v[r, pl.ds(q * 16, 16)] = zrow
        return 0

    lax.fori_loop(0, CH, fill, 0)

    pltpu.sync_copy(ei_hbm.at[0, pl.ds(wid * CPW, CPW)], idx_v)

    base = s * ZROWS
    for k in range(ZROWS // CH):
        pltpu.sync_copy(zero_v, deg_sh.at[pl.ds(base + k * CH, CH)])
    plsc.subcore_barrier()

    def scat(j, _):
        pltpu.sync_copy(ones_v, deg_sh.at[idx_v.at[j]], add=True)
        return 0

    lax.fori_loop(0, CPW, scat, 0)
    plsc.subcore_barrier()
    pltpu.sync_copy(deg_sh.at[pl.ds(base, ZROWS)],
                    deg_out.at[c, pl.ds(base, ZROWS)])


_hist = pl.kernel(
    _hist_body,
    out_type=jax.ShapeDtypeStruct((NC, NPAD, HW), jnp.float32),
    mesh=_mesh,
    scratch_types=[
        pltpu.VMEM_SHARED((NPAD, HW), jnp.float32),
        pltpu.VMEM((CPW, CH), jnp.int32),
        pltpu.VMEM((CH, HW), jnp.float32),
        pltpu.VMEM((CH, HW), jnp.float32),
    ],
)


def _agg_body(xs_hbm, ei_hbm, part_out,
              agg_sh, rows_v, cols_v, gbuf, s0, s1, s2, s3):
    c = lax.axis_index("c")
    s = lax.axis_index("s")
    wid = c * NS + s
    cb = wid * CPW

    zrow = jnp.zeros((16,), jnp.float32)

    def fill(r, _):
        for q in range(D // 16):
            gbuf[0, r, pl.ds(q * 16, 16)] = zrow
        return 0

    lax.fori_loop(0, CH, fill, 0)

    pltpu.sync_copy(ei_hbm.at[0, pl.ds(cb, CPW)], rows_v)
    pltpu.sync_copy(ei_hbm.at[1, pl.ds(cb, CPW)], cols_v)

    def clamp(j, _):
        for q in range(CH // 16):
            v = cols_v[j, pl.ds(q * 16, 16)]
            dumped = DUMP + ((v - DUMP) & 63)
            cols_v[j, pl.ds(q * 16, 16)] = jnp.where(v < DUMP, v, dumped)
        return 0

    lax.fori_loop(0, CPW, clamp, 0)

    base = s * ZAGG
    for k in range(ZAGG // CH):
        pltpu.sync_copy(gbuf.at[0], agg_sh.at[pl.ds(base + k * CH, CH)])
    rem = ZAGG % CH
    if rem:
        pltpu.sync_copy(gbuf.at[0, pl.ds(0, rem)],
                        agg_sh.at[pl.ds(base + (ZAGG // CH) * CH, rem)])
    plsc.subcore_barrier()

    def start(j, b, sem):
        pltpu.async_copy(xs_hbm.at[rows_v.at[j]], gbuf.at[b], sem)

    def wait(b, sem):
        pltpu.make_async_copy(xs_hbm.at[rows_v.at[0]], gbuf.at[b], sem).wait()

    def scat(j, b):
        pltpu.sync_copy(gbuf.at[b], agg_sh.at[cols_v.at[j]], add=True)

    sems = (s0, s1, s2, s3)
    for b in range(4):
        start(b, b, sems[b])

    def body(g, _):
        j0 = 4 * g
        for b in range(4):
            wait(b, sems[b])
            scat(j0 + b, b)
            start(j0 + b + 4, b, sems[b])
        return 0

    lax.fori_loop(0, CPW // 4 - 1, body, 0)
    for b in range(4):
        wait(b, sems[b])
        scat(CPW - 4 + b, b)

    plsc.subcore_barrier()
    pltpu.sync_copy(agg_sh.at[pl.ds(s * OROWS, OROWS)],
                    part_out.at[c, pl.ds(s * OROWS, OROWS)])


_agg = pl.kernel(
    _agg_body,
    out_type=jax.ShapeDtypeStruct((NC, OPAD, D), jnp.float32),
    mesh=_mesh,
    scratch_types=[
        pltpu.VMEM_SHARED((NAGG, D), jnp.float32),
        pltpu.VMEM((CPW, CH), jnp.int32),
        pltpu.VMEM((CPW, CH), jnp.int32),
        pltpu.VMEM((4, CH, D), jnp.float32),
        pltpu.SemaphoreType.DMA,
        pltpu.SemaphoreType.DMA,
        pltpu.SemaphoreType.DMA,
        pltpu.SemaphoreType.DMA,
    ],
)


def _prescale_body(deg_ref, x_ref, xs_ref):
    deg = deg_ref[0, 0:N, 0:1] + deg_ref[1, 0:N, 0:1]
    dinv = jnp.where(deg > 0, lax.rsqrt(deg), 0.0)
    xs_ref[0:N, :] = x_ref[...] * dinv
    xs_ref[N:, :] = jnp.zeros((NPAD - N, D), jnp.float32)


def _finalize_body(x_ref, deg_ref, part_ref, out_ref):
    deg = deg_ref[0, 0:5000, 0:1] + deg_ref[1, 0:5000, 0:1]
    dinv = jnp.where(deg > 0, lax.rsqrt(deg), 0.0)
    agg = part_ref[0, 0:5000, :] + part_ref[1, 0:5000, :]
    out_ref[:, 0:D] = x_ref[0:5000, :]
    out_ref[:, D:] = agg * dinv


def kernel(x, edge_index, batch_size):
    del batch_size
    arange_pad = jnp.arange(EPAD - E, dtype=jnp.int32)
    pad = jnp.stack([N + arange_pad % (NPAD - N), N + arange_pad % 64])
    ei = jnp.concatenate([edge_index, pad], axis=1).reshape(2, NCHUNK, CH)

    deg2 = _hist(ei)
    xs = pl.pallas_call(
        _prescale_body,
        out_shape=jax.ShapeDtypeStruct((NPAD, D), jnp.float32),
    )(deg2, x)
    part = _agg(xs, ei)
    out = pl.pallas_call(
        _finalize_body,
        out_shape=jax.ShapeDtypeStruct((5000, 2 * D), jnp.float32),
    )(x, deg2, part)
    return out
